# baseline (speedup 1.0000x reference)
import functools

import jax
import jax.numpy as jnp
from jax import lax
from jax.experimental import pallas as pl
from jax.experimental.pallas import tpu as pltpu
from jax.experimental.pallas import tpu_sc as plsc

_VOCAB = 1000
_B, _T = 1024, 50
_N = _B * _T
_NW = 32
_ROWS_PER_W = _N // _NW
_CHUNK = 32
_NCHUNKS = _ROWS_PER_W // _CHUNK


def _sc_gather(idx_grouped, table):
    mesh = plsc.VectorSubcoreMesh(core_axis_name="c", subcore_axis_name="s")

    @functools.partial(
        pl.kernel,
        mesh=mesh,
        out_type=jax.ShapeDtypeStruct((_N, _VOCAB), jnp.float32),
        scratch_types=[
            pltpu.VMEM((_NCHUNKS, _CHUNK), jnp.int32),
            pltpu.VMEM((_CHUNK, _VOCAB), jnp.float32),
            pltpu.SemaphoreType.DMA,
        ],
        compiler_params=pltpu.CompilerParams(use_tc_tiling_on_sc=False),
    )
    def k(idx_hbm, table_hbm, out_hbm, idx_v, buf, sem):
        wid = lax.axis_index("s") * 2 + lax.axis_index("c")
        base = wid * _ROWS_PER_W
        pltpu.sync_copy(idx_hbm.at[wid], idx_v)

        @pl.loop(0, _NCHUNKS)
        def _(c):
            pltpu.async_copy(table_hbm.at[idx_v.at[c]], buf, sem).wait()
            pltpu.sync_copy(buf, out_hbm.at[pl.ds(base + c * _CHUNK, _CHUNK)])

    return k(idx_grouped, table)


def kernel(idx, table):
    idx_grouped = idx.reshape(_NW, _NCHUNKS, _CHUNK).astype(jnp.int32)
    out = _sc_gather(idx_grouped, table)
    return out.reshape(_B, _T, _VOCAB)


# speedup vs baseline: 1.0490x; 1.0490x over previous
import functools

import jax
import jax.numpy as jnp
from jax import lax
from jax.experimental import pallas as pl
from jax.experimental.pallas import tpu as pltpu
from jax.experimental.pallas import tpu_sc as plsc

_VOCAB = 1000
_B, _T = 1024, 50
_N = _B * _T
_NW = 32
_ROWS_PER_W = _N // _NW
_CHUNK = 50
_NCHUNKS = _ROWS_PER_W // _CHUNK


def _sc_gather(idx_grouped, table):
    mesh = plsc.VectorSubcoreMesh(core_axis_name="c", subcore_axis_name="s")

    @functools.partial(
        pl.kernel,
        mesh=mesh,
        out_type=jax.ShapeDtypeStruct((_N, _VOCAB), jnp.float32),
        scratch_types=[
            pltpu.VMEM((_NCHUNKS, _CHUNK), jnp.int32),
            pltpu.VMEM((_CHUNK, _VOCAB), jnp.float32),
            pltpu.VMEM((_CHUNK, _VOCAB), jnp.float32),
            pltpu.SemaphoreType.DMA,
            pltpu.SemaphoreType.DMA,
            pltpu.SemaphoreType.DMA,
            pltpu.SemaphoreType.DMA,
        ],
        compiler_params=pltpu.CompilerParams(use_tc_tiling_on_sc=False),
    )
    def k(idx_hbm, table_hbm, out_hbm, idx_v, buf0, buf1, g0, g1, s0, s1):
        bufs = (buf0, buf1)
        gsems = (g0, g1)
        ssems = (s0, s1)
        wid = lax.axis_index("s") * 2 + lax.axis_index("c")
        base = wid * _ROWS_PER_W
        pltpu.sync_copy(idx_hbm.at[wid], idx_v)

        def out_rows(c):
            return out_hbm.at[pl.ds(base + c * _CHUNK, _CHUNK)]

        pltpu.async_copy(table_hbm.at[idx_v.at[0]], buf0, g0)
        pltpu.async_copy(table_hbm.at[idx_v.at[1]], buf1, g1)

        @pl.loop(0, _NCHUNKS - 2, step=2)
        def _(j):
            for b in range(2):
                c = j + b
                pltpu.make_async_copy(
                    table_hbm.at[idx_v.at[c]], bufs[b], gsems[b]).wait()
                pltpu.async_copy(bufs[b], out_rows(c), ssems[b])
                pltpu.make_async_copy(bufs[b], out_rows(c), ssems[b]).wait()
                pltpu.async_copy(
                    table_hbm.at[idx_v.at[c + 2]], bufs[b], gsems[b])

        for b in range(2):
            c = _NCHUNKS - 2 + b
            pltpu.make_async_copy(
                table_hbm.at[idx_v.at[c]], bufs[b], gsems[b]).wait()
            pltpu.sync_copy(bufs[b], out_rows(c))

    return k(idx_grouped, table)


def kernel(idx, table):
    idx_grouped = idx.reshape(_NW, _NCHUNKS, _CHUNK).astype(jnp.int32)
    out = _sc_gather(idx_grouped, table)
    return out.reshape(_B, _T, _VOCAB)


# speedup vs baseline: 1.0503x; 1.0012x over previous
import functools

import jax
import jax.numpy as jnp
from jax import lax
from jax.experimental import pallas as pl
from jax.experimental.pallas import tpu as pltpu
from jax.experimental.pallas import tpu_sc as plsc

_VOCAB = 1000
_B, _T = 1024, 50
_NW = 32
_BPW = _B // _NW
_CHUNK = _T


def _sc_gather(idx, table):
    mesh = plsc.VectorSubcoreMesh(core_axis_name="c", subcore_axis_name="s")

    @functools.partial(
        pl.kernel,
        mesh=mesh,
        out_type=jax.ShapeDtypeStruct((_B, _T, _VOCAB), jnp.float32),
        scratch_types=[
            pltpu.VMEM((_BPW, _CHUNK), jnp.int32),
            pltpu.VMEM((_CHUNK, _VOCAB), jnp.float32),
            pltpu.VMEM((_CHUNK, _VOCAB), jnp.float32),
            pltpu.SemaphoreType.DMA,
            pltpu.SemaphoreType.DMA,
            pltpu.SemaphoreType.DMA,
            pltpu.SemaphoreType.DMA,
        ],
        compiler_params=pltpu.CompilerParams(use_tc_tiling_on_sc=False),
    )
    def k(idx_hbm, table_hbm, out_hbm, idx_v, buf0, buf1, g0, g1, s0, s1):
        bufs = (buf0, buf1)
        gsems = (g0, g1)
        ssems = (s0, s1)
        wid = lax.axis_index("s") * 2 + lax.axis_index("c")
        base = wid * _BPW
        pltpu.sync_copy(idx_hbm.at[pl.ds(base, _BPW)], idx_v)

        pltpu.async_copy(table_hbm.at[idx_v.at[0]], buf0, g0)
        pltpu.async_copy(table_hbm.at[idx_v.at[1]], buf1, g1)

        @pl.loop(0, _BPW - 2, step=2)
        def _(j):
            for b in range(2):
                c = j + b
                pltpu.make_async_copy(
                    table_hbm.at[idx_v.at[c]], bufs[b], gsems[b]).wait()
                pltpu.async_copy(bufs[b], out_hbm.at[base + c], ssems[b])
                pltpu.make_async_copy(
                    bufs[b], out_hbm.at[base + c], ssems[b]).wait()
                pltpu.async_copy(
                    table_hbm.at[idx_v.at[c + 2]], bufs[b], gsems[b])

        for b in range(2):
            c = _BPW - 2 + b
            pltpu.make_async_copy(
                table_hbm.at[idx_v.at[c]], bufs[b], gsems[b]).wait()
            pltpu.sync_copy(bufs[b], out_hbm.at[base + c])

    return k(idx, table)


def kernel(idx, table):
    return _sc_gather(idx.astype(jnp.int32), table)


# speedup vs baseline: 2.1144x; 2.0131x over previous
import functools

import jax
import jax.numpy as jnp
from jax import lax
from jax.experimental import pallas as pl
from jax.experimental.pallas import tpu as pltpu
from jax.experimental.pallas import tpu_sc as plsc

_VOCAB = 1000
_VPAD = 1024
_B, _T = 1024, 50
_NW = 32
_BPW = _B // _NW


def _sc_gather(idx_p, table_p):
    mesh = plsc.VectorSubcoreMesh(core_axis_name="c", subcore_axis_name="s")

    @functools.partial(
        pl.kernel,
        mesh=mesh,
        out_type=jax.ShapeDtypeStruct((_B, _T, _VPAD), jnp.float32),
        scratch_types=[
            pltpu.VMEM((8, _T), jnp.int32),
            pltpu.VMEM((8, _T), jnp.int32),
            pltpu.VMEM((_T, _VPAD), jnp.float32),
            pltpu.VMEM((_T, _VPAD), jnp.float32),
            pltpu.SemaphoreType.DMA,
            pltpu.SemaphoreType.DMA,
            pltpu.SemaphoreType.DMA,
            pltpu.SemaphoreType.DMA,
            pltpu.SemaphoreType.DMA,
            pltpu.SemaphoreType.DMA,
        ],
        compiler_params=pltpu.CompilerParams(use_tc_tiling_on_sc=True),
    )
    def k(idx_hbm, table_hbm, out_hbm, i0, i1, buf0, buf1,
          gi0, gi1, g0, g1, s0, s1):
        ibufs = (i0, i1)
        bufs = (buf0, buf1)
        isems = (gi0, gi1)
        gsems = (g0, g1)
        ssems = (s0, s1)
        wid = lax.axis_index("s") * 2 + lax.axis_index("c")
        base = wid * _BPW

        for b in range(2):
            pltpu.sync_copy(idx_hbm.at[base + b], ibufs[b])
            pltpu.async_copy(table_hbm.at[ibufs[b].at[0]], bufs[b], gsems[b])

        @pl.loop(0, _BPW - 2, step=2)
        def _(j):
            for b in range(2):
                c = j + b
                pltpu.make_async_copy(
                    table_hbm.at[ibufs[b].at[0]], bufs[b], gsems[b]).wait()
                pltpu.async_copy(bufs[b], out_hbm.at[base + c], ssems[b])
                pltpu.async_copy(idx_hbm.at[base + c + 2], ibufs[b], isems[b])
                pltpu.make_async_copy(
                    bufs[b], out_hbm.at[base + c], ssems[b]).wait()
                pltpu.make_async_copy(
                    idx_hbm.at[base + c + 2], ibufs[b], isems[b]).wait()
                pltpu.async_copy(
                    table_hbm.at[ibufs[b].at[0]], bufs[b], gsems[b])

        for b in range(2):
            c = _BPW - 2 + b
            pltpu.make_async_copy(
                table_hbm.at[ibufs[b].at[0]], bufs[b], gsems[b]).wait()
            pltpu.sync_copy(bufs[b], out_hbm.at[base + c])

    return k(idx_p, table_p)


def kernel(idx, table):
    idx_p = jnp.pad(idx.reshape(_B, 1, _T).astype(jnp.int32),
                    ((0, 0), (0, 7), (0, 0)))
    table_p = jnp.pad(table, ((0, 0), (0, _VPAD - _VOCAB)))
    out = _sc_gather(idx_p, table_p)
    return out[:, :, :_VOCAB]
